# baseline (speedup 1.0000x reference)
import functools

import jax
import jax.numpy as jnp
from jax import lax
from jax.experimental import pallas as pl
from jax.experimental.pallas import tpu as pltpu
from jax.experimental.pallas import tpu_sc as plsc

_INFO = plsc.get_sparse_core_info()
_NC, _NS = _INFO.num_cores, _INFO.num_subcores
_NW = _NC * _NS
_CH = 128


def _build(N, V, D, L, B):
    n_chunks = N // _CH
    pw = n_chunks // _NW
    kd = D // 8
    mb = B // _CH
    mesh = plsc.VectorSubcoreMesh(core_axis_name="c", subcore_axis_name="s")

    @functools.partial(
        pl.kernel,
        out_type=jax.ShapeDtypeStruct((L, kd, mb, 8, _CH), jnp.float32),
        mesh=mesh,
        compiler_params=pltpu.CompilerParams(
            use_tc_tiling_on_sc=False, needs_layout_passes=False),
        scratch_types=[
            pltpu.VMEM((pw, _CH), jnp.int32),
            pltpu.VMEM((pw, _CH), jnp.int32),
            pltpu.VMEM((_CH, D), jnp.float32),
            pltpu.VMEM((_CH, D), jnp.float32),
            pltpu.VMEM((kd, 8, _CH), jnp.float32),
            pltpu.VMEM((kd, 8, _CH), jnp.float32),
            pltpu.SemaphoreType.DMA,
            pltpu.SemaphoreType.DMA,
            pltpu.SemaphoreType.DMA,
            pltpu.SemaphoreType.DMA,
            pltpu.SemaphoreType.DMA,
        ],
    )
    def two_level_gather(batch_hbm, gmap_hbm, emb_hbm, out_hbm,
                         idx_v, gids_v, g0, g1, t0, t1,
                         sem_g, sem_r0, sem_r1, sem_w0, sem_w1):
        wid = lax.axis_index("s") * _NC + lax.axis_index("c")
        chunk_base = wid * pw

        pltpu.sync_copy(batch_hbm.at[wid], idx_v)

        def fire_remap(j, carry):
            pltpu.async_copy(gmap_hbm.at[idx_v.at[j]], gids_v.at[j], sem_g)
            return carry
        lax.fori_loop(0, pw, fire_remap, 0)
        pltpu.make_async_copy(batch_hbm.at[wid], gids_v, sem_g).wait()

        gbufs = (g0, g1)
        tbufs = (t0, t1)
        rsems = (sem_r0, sem_r1)
        wsems = (sem_w0, sem_w1)

        iota = lax.broadcasted_iota(jnp.int32, (16,), 0)
        rows16 = [iota + b0 for b0 in range(0, _CH, 16)]

        def transpose_chunk(g, t):
            def body(k, carry):
                for d8 in range(8):
                    col = jnp.zeros((16,), jnp.int32) + (k * 8 + d8)
                    for i, r16 in enumerate(rows16):
                        v = plsc.load_gather(g, [r16, col])
                        t[k, d8, pl.ds(i * 16, 16)] = v
                return carry
            lax.fori_loop(0, kd, body, 0)

        def lm(j):
            c = chunk_base + j
            return c // mb, c % mb

        def fire_gather(j, b):
            pltpu.async_copy(emb_hbm.at[gids_v.at[j]], gbufs[b], rsems[b])

        def writeback(j, b):
            l, m = lm(j)
            pltpu.async_copy(tbufs[b], out_hbm.at[l, :, m], wsems[b])

        def wait_gather(j, b):
            pltpu.make_async_copy(emb_hbm.at[gids_v.at[j]], gbufs[b],
                                  rsems[b]).wait()

        def wait_writeback(j, b):
            l, m = lm(j)
            pltpu.make_async_copy(tbufs[b], out_hbm.at[l, :, m],
                                  wsems[b]).wait()

        fire_gather(0, 0)
        fire_gather(1, 1)
        for b in range(2):
            wait_gather(b, b)
            transpose_chunk(gbufs[b], tbufs[b])
            writeback(b, b)
            fire_gather(b + 2, b)

        def body(gi, carry):
            for b in range(2):
                j = 2 * gi + b
                wait_gather(j, b)
                wait_writeback(j - 2, b)
                transpose_chunk(gbufs[b], tbufs[b])
                writeback(j, b)
                fire_gather(j + 2, b)
            return carry
        lax.fori_loop(1, pw // 2 - 1, body, 0)

        for b in range(2):
            j = pw - 2 + b
            wait_gather(j, b)
            wait_writeback(j - 2, b)
            transpose_chunk(gbufs[b], tbufs[b])
            writeback(j, b)
        for b in range(2):
            wait_writeback(pw - 2 + b, b)

    return two_level_gather


def kernel(batch, glove_id_map, embeddings):
    B, L = batch.shape
    V, D = embeddings.shape
    N = B * L
    idx3d = batch.T.reshape(_NW, N // (_NW * _CH), _CH).astype(jnp.int32)
    o5 = _build(N, V, D, L, B)(idx3d, glove_id_map, embeddings)
    return o5.transpose(2, 4, 0, 1, 3).reshape(B, L, D)


# speedup vs baseline: 1.2575x; 1.2575x over previous
import functools

import jax
import jax.numpy as jnp
from jax import lax
from jax.experimental import pallas as pl
from jax.experimental.pallas import tpu as pltpu
from jax.experimental.pallas import tpu_sc as plsc

_INFO = plsc.get_sparse_core_info()
_NC, _NS = _INFO.num_cores, _INFO.num_subcores
_NW = _NC * _NS
_CH = 128


def _build(N, V, D, L, B):
    n_chunks = N // _CH
    pw = n_chunks // _NW
    kd = D // 8
    mb = B // _CH
    mesh = plsc.VectorSubcoreMesh(core_axis_name="c", subcore_axis_name="s")

    @functools.partial(
        pl.kernel,
        out_type=jax.ShapeDtypeStruct((L, kd, mb, 8, _CH), jnp.float32),
        mesh=mesh,
        compiler_params=pltpu.CompilerParams(
            use_tc_tiling_on_sc=False, needs_layout_passes=False),
        scratch_types=[
            pltpu.VMEM((pw, _CH), jnp.int32),
            pltpu.VMEM((pw, _CH), jnp.int32),
            pltpu.VMEM((_CH, D), jnp.float32),
            pltpu.VMEM((_CH, D), jnp.float32),
            pltpu.VMEM((kd, 8, _CH), jnp.float32),
            pltpu.VMEM((kd, 8, _CH), jnp.float32),
            pltpu.SemaphoreType.DMA,
            pltpu.SemaphoreType.DMA,
            pltpu.SemaphoreType.DMA,
            pltpu.SemaphoreType.DMA,
            pltpu.SemaphoreType.DMA,
        ],
    )
    def two_level_gather(batch_hbm, gmap_hbm, emb_hbm, out_hbm,
                         idx_v, gids_v, g0, g1, t0, t1,
                         sem_g, sem_r0, sem_r1, sem_w0, sem_w1):
        wid = lax.axis_index("s") * _NC + lax.axis_index("c")
        chunk_base = wid * pw

        pltpu.sync_copy(batch_hbm.at[wid], idx_v)

        def fire_remap(j, carry):
            pltpu.async_copy(gmap_hbm.at[idx_v.at[j]], gids_v.at[j], sem_g)
            return carry
        lax.fori_loop(0, pw, fire_remap, 0)
        pltpu.make_async_copy(batch_hbm.at[wid], gids_v, sem_g).wait()

        gbufs = (g0, g1)
        tbufs = (t0, t1)
        rsems = (sem_r0, sem_r1)
        wsems = (sem_w0, sem_w1)

        iota = lax.broadcasted_iota(jnp.int32, (16,), 0)
        rows16 = [iota + b0 for b0 in range(0, _CH, 16)]

        def transpose_chunk(g, t):
            @plsc.parallel_loop(0, kd, unroll=2)
            def _(k):
                for d8 in range(8):
                    col = jnp.zeros((16,), jnp.int32) + (k * 8 + d8)
                    for i, r16 in enumerate(rows16):
                        v = plsc.load_gather(g, [r16, col])
                        t[k, d8, pl.ds(i * 16, 16)] = v

        def lm(j):
            c = chunk_base + j
            return c // mb, c % mb

        def fire_gather(j, b):
            pltpu.async_copy(emb_hbm.at[gids_v.at[j]], gbufs[b], rsems[b])

        def writeback(j, b):
            l, m = lm(j)
            pltpu.async_copy(tbufs[b], out_hbm.at[l, :, m], wsems[b])

        def wait_gather(j, b):
            pltpu.make_async_copy(emb_hbm.at[gids_v.at[j]], gbufs[b],
                                  rsems[b]).wait()

        def wait_writeback(j, b):
            l, m = lm(j)
            pltpu.make_async_copy(tbufs[b], out_hbm.at[l, :, m],
                                  wsems[b]).wait()

        fire_gather(0, 0)
        fire_gather(1, 1)
        for b in range(2):
            wait_gather(b, b)
            transpose_chunk(gbufs[b], tbufs[b])
            writeback(b, b)
            fire_gather(b + 2, b)

        def body(gi, carry):
            for b in range(2):
                j = 2 * gi + b
                wait_gather(j, b)
                wait_writeback(j - 2, b)
                transpose_chunk(gbufs[b], tbufs[b])
                writeback(j, b)
                fire_gather(j + 2, b)
            return carry
        lax.fori_loop(1, pw // 2 - 1, body, 0)

        for b in range(2):
            j = pw - 2 + b
            wait_gather(j, b)
            wait_writeback(j - 2, b)
            transpose_chunk(gbufs[b], tbufs[b])
            writeback(j, b)
        for b in range(2):
            wait_writeback(pw - 2 + b, b)

    return two_level_gather


def kernel(batch, glove_id_map, embeddings):
    B, L = batch.shape
    V, D = embeddings.shape
    N = B * L
    idx3d = batch.T.reshape(_NW, N // (_NW * _CH), _CH).astype(jnp.int32)
    o5 = _build(N, V, D, L, B)(idx3d, glove_id_map, embeddings)
    return o5.transpose(2, 4, 0, 1, 3).reshape(B, L, D)
